# Initial kernel scaffold; baseline (speedup 1.0000x reference)
import jax
import jax.numpy as jnp
from jax.experimental import pallas as pl


def kernel(vessel_labels):
    raise NotImplementedError("write your pallas kernel here")


# speedup vs baseline: 97.3855x; 97.3855x over previous
import functools

import numpy as np
import jax
import jax.numpy as jnp
from jax import lax
from jax.experimental import pallas as pl
from jax.experimental.pallas import tpu as pltpu
from jax.experimental.pallas import tpu_sc as plsc

_D = 128
_SLICE = 128 * 128
_NVOX = _D * _SLICE
_NW = 32
_L = 16
_VPW = _NVOX // _NW
_PCHUNK = 16384
_NIDS = 48
_TAB = 64


_F0, _F1, _F2 = True, False, True
_SUBN_DATA = np.array([3647288517, 4265293960], np.uint32)
_KEY0_DATA = np.array([1889313301, 2441599006], np.uint32)
_PERM = np.array([11, 38, 5, 16, 46, 45, 47, 7, 39, 15, 1, 2, 40, 8, 43, 27,
                  24, 32, 23, 36, 10, 28, 37, 42, 35, 14, 17, 13, 0, 9, 6, 12,
                  25, 41, 34, 19, 3, 20, 44, 4, 31, 22, 33, 30, 29, 26, 21,
                  18], np.int32)


def _presence_body(labels_hbm, out_hbm, labs_v, pres_v):
    c = lax.axis_index("c")
    s = lax.axis_index("s")
    wid = s * 2 + c
    base = wid * _VPW
    zeros = jnp.zeros((_L,), jnp.float32)
    for i in range(_TAB // _L):
        pres_v[pl.ds(i * _L, _L)] = zeros
    ones = jnp.ones((_L,), jnp.float32)

    def chunk_body(ci, carry):
        pltpu.sync_copy(labels_hbm.at[pl.ds(base + ci * _PCHUNK, _PCHUNK)],
                        labs_v)

        def grp(gi, carry2):
            labs = labs_v[pl.ds(gi * _L, _L)]
            plsc.store_scatter(pres_v, [labs], ones)
            return carry2

        return lax.fori_loop(0, _PCHUNK // _L, grp, carry)

    lax.fori_loop(0, _VPW // _PCHUNK, chunk_body, 0)
    pltpu.sync_copy(pres_v, out_hbm.at[wid])


def _tables(present):
    n = jnp.sum(present)
    sub_n = jax.random.wrap_key_data(jnp.asarray(_SUBN_DATA))
    n_hide = jax.random.randint(sub_n, (), n // 2, n - 1)
    hidden = jnp.zeros(_NIDS, bool).at[jnp.asarray(_PERM)].set(
        jnp.arange(_NIDS) < n_hide)
    kept = present & (~hidden)
    key0 = jax.random.wrap_key_data(jnp.asarray(_KEY0_DATA))

    def step(key, keptu):
        def _draw(k):
            k, k1, k2 = jax.random.split(k, 3)
            b = jax.random.bernoulli(k1, 0.5)
            lo = jnp.where(b, 0.0, 1.9)
            hi = jnp.where(b, 0.1, 2.0)
            val = jax.random.uniform(k2, (), minval=lo, maxval=hi)
            return k, b, val

        def _skip(k):
            return k, jnp.zeros((), bool), jnp.zeros((), jnp.result_type(float))

        key, b, val = lax.cond(keptu, _draw, _skip, key)
        return key, (b, val)

    _, (bs, vals) = lax.scan(step, key0, kept[1:_NIDS])
    b_full = jnp.concatenate([jnp.zeros(1, bool), bs])
    v_full = jnp.concatenate([jnp.zeros(1, vals.dtype), vals])
    u = jnp.arange(_NIDS)
    m = kept & (u >= 1)
    scal_tab = jnp.where(m, v_full, 1.0).astype(jnp.float32)
    oh1_tab = jnp.where(m & b_full, 1.0, 0.0).astype(jnp.float32)
    oh2_tab = jnp.where(m & (~b_full), 1.0, 0.0).astype(jnp.float32)
    bg_tab = jnp.where(m, 0.0, 1.0).astype(jnp.float32)
    tabs = jnp.zeros(4 * _TAB, jnp.float32)
    tabs = tabs.at[0:_NIDS].set(scal_tab)
    tabs = tabs.at[_TAB:_TAB + _NIDS].set(oh1_tab)
    tabs = tabs.at[2 * _TAB:2 * _TAB + _NIDS].set(oh2_tab)
    tabs = tabs.at[3 * _TAB:3 * _TAB + _NIDS].set(bg_tab)
    return tabs


def _src_index(g, iota):
    r = g >> 3
    cc = g & 7
    ir = (127 - r) if _F1 else r
    if _F2:
        return (ir * 128 + 127 - cc * _L) - iota
    return (ir * 128 + cc * _L) + iota


def _fill_body(lab_hbm, tabs_hbm, scal_hbm, oh_hbm,
               labs_v, tabs_v, s_v, o1_v, o2_v, bg_v):
    c = lax.axis_index("c")
    s = lax.axis_index("s")
    wid = s * 2 + c
    pltpu.sync_copy(tabs_hbm, tabs_v)
    iota = lax.iota(jnp.int32, _L)

    di0 = (_D - 1) if _F0 else 0
    pltpu.sync_copy(lab_hbm.at[di0], labs_v)

    def bg_body(g, carry):
        labs = plsc.load_gather(labs_v, [_src_index(g, iota)])
        bgv = plsc.load_gather(tabs_v, [labs + (3 * _TAB)])
        bg_v[pl.ds(g * _L, _L)] = bgv
        return carry

    lax.fori_loop(0, _SLICE // _L, bg_body, 0)

    for k in range(_D // _NW):
        d = wid * (_D // _NW) + k
        di = (127 - d) if _F0 else d
        pltpu.sync_copy(lab_hbm.at[di], labs_v)

        def g_body(g, carry):
            labs = plsc.load_gather(labs_v, [_src_index(g, iota)])
            sv = plsc.load_gather(tabs_v, [labs])
            o1 = plsc.load_gather(tabs_v, [labs + _TAB])
            o2 = plsc.load_gather(tabs_v, [labs + (2 * _TAB)])
            q = g * _L
            s_v[pl.ds(q, _L)] = sv
            o1_v[pl.ds(q, _L)] = o1
            o2_v[pl.ds(q, _L)] = o2
            return carry

        lax.fori_loop(0, _SLICE // _L, g_body, 0)
        pltpu.sync_copy(s_v, scal_hbm.at[d])
        pltpu.sync_copy(o1_v, oh_hbm.at[1, d])
        pltpu.sync_copy(o2_v, oh_hbm.at[2, d])
        pltpu.sync_copy(bg_v, oh_hbm.at[0, d])


@functools.lru_cache(maxsize=None)
def _build_kernels():
    mesh = plsc.VectorSubcoreMesh(
        core_axis_name="c", subcore_axis_name="s",
        num_cores=2, num_subcores=16)
    params = pltpu.CompilerParams(needs_layout_passes=False)
    presence = pl.kernel(
        _presence_body,
        out_type=jax.ShapeDtypeStruct((_NW, _TAB), jnp.float32),
        mesh=mesh,
        scratch_types=[
            pltpu.VMEM((_PCHUNK,), jnp.int32),
            pltpu.VMEM((_TAB,), jnp.float32),
        ],
        compiler_params=params,
    )
    fill = pl.kernel(
        _fill_body,
        out_type=(
            jax.ShapeDtypeStruct((_D, _SLICE), jnp.float32),
            jax.ShapeDtypeStruct((3, _D, _SLICE), jnp.float32),
        ),
        mesh=mesh,
        scratch_types=[
            pltpu.VMEM((_SLICE,), jnp.int32),
            pltpu.VMEM((4 * _TAB,), jnp.float32),
            pltpu.VMEM((_SLICE,), jnp.float32),
            pltpu.VMEM((_SLICE,), jnp.float32),
            pltpu.VMEM((_SLICE,), jnp.float32),
            pltpu.VMEM((_SLICE,), jnp.float32),
        ],
        compiler_params=params,
    )
    return presence, fill


def kernel(vessel_labels):
    presence_kernel, fill_kernel = _build_kernels()
    flat = vessel_labels.reshape(_NVOX)
    pres_part = presence_kernel(flat)
    present = jnp.any(pres_part[:, :_NIDS] > 0.0, axis=0)
    tabs = _tables(present)
    lab2 = vessel_labels.reshape(_D, _SLICE)
    scal, oh = fill_kernel(lab2, tabs)
    return scal.reshape(_D, 128, 128), oh.reshape(3, _D, 128, 128)
